# baseline (speedup 1.0000x reference)
import functools

import jax
import jax.numpy as jnp
from jax import lax
from jax.experimental import pallas as pl
from jax.experimental.pallas import tpu as pltpu
from jax.experimental.pallas import tpu_sc as plsc

DEPTH = 63
BATCH = 16384
NUM_WORKERS = 16
ROWS = BATCH // NUM_WORKERS
NCHUNK = 8
CROWS = ROWS // NCHUNK

_TABLE = jnp.eye(DEPTH, dtype=jnp.float32)

_mesh = plsc.VectorSubcoreMesh(core_axis_name="c", subcore_axis_name="s",
                               num_cores=1)


@functools.partial(
    pl.kernel,
    mesh=_mesh,
    out_type=jax.ShapeDtypeStruct((BATCH, DEPTH), jnp.float32),
    scratch_types=[
        pltpu.VMEM((NCHUNK, CROWS), jnp.int32),
        pltpu.VMEM((ROWS, DEPTH), jnp.float32),
        pltpu.SemaphoreType.DMA,
        pltpu.SemaphoreType.DMA,
    ],
    compiler_params=pltpu.CompilerParams(
        needs_layout_passes=False,
        use_tc_tiling_on_sc=False,
        skip_device_barrier=True,
        disable_bounds_checks=True,
        disable_semaphore_checks=True,
    ),
)
def _one_hot_sc(idx_hbm, table_hbm, out_hbm, idx_v, rows_v, sem_g, sem_out):
    wid = lax.axis_index("s")
    row_base = wid * ROWS

    for c in range(NCHUNK):
        pltpu.sync_copy(idx_hbm.at[pl.ds(row_base + c * CROWS, CROWS)],
                        idx_v.at[c])

    def gather(c):
        return pltpu.async_copy(
            table_hbm.at[idx_v.at[c]],
            rows_v.at[pl.ds(c * CROWS, CROWS)],
            sem_g,
        )

    pending = gather(0)
    copies = []
    for c in range(NCHUNK):
        pending.wait()
        if c + 1 < NCHUNK:
            pending = gather(c + 1)
        copies.append(
            pltpu.async_copy(
                rows_v.at[pl.ds(c * CROWS, CROWS)],
                out_hbm.at[pl.ds(row_base + c * CROWS, CROWS)],
                sem_out,
            )
        )
    for cp in copies:
        cp.wait()


def kernel(indices):
    return _one_hot_sc(indices, _TABLE)
